# Initial kernel scaffold; baseline (speedup 1.0000x reference)
import jax
import jax.numpy as jnp
from jax.experimental import pallas as pl


def kernel(t, x, edge_index, W1, W2):
    raise NotImplementedError("write your pallas kernel here")


# speedup vs baseline: 15.2176x; 15.2176x over previous
import functools

import jax
import jax.numpy as jnp
from jax import lax
from jax.experimental import pallas as pl
from jax.experimental.pallas import tpu as pltpu
from jax.experimental.pallas import tpu_sc as plsc

N = 10000
E = 320000
D = 128

NC = 2
NS = 16
NW = NC * NS
EPW = E // NW
K = 80
C = EPW // K
N_PAD = 10240
RSUB = N_PAD // NS

_MESH = plsc.VectorSubcoreMesh(core_axis_name="c", subcore_axis_name="s")


def _sc_degree(dst_r, ones_hbm, zeros_hbm):
    @functools.partial(
        pl.kernel,
        out_type=jax.ShapeDtypeStruct((NC, N_PAD, D), jnp.float32),
        mesh=_MESH,
        scratch_types=[
            pltpu.VMEM((C, K), jnp.int32),
            pltpu.VMEM((K, D), jnp.float32),
            pltpu.VMEM_SHARED((N_PAD, D), jnp.float32),
        ],
    )
    def deg_kernel(dst_hbm, ones_h, zeros_h, out_hbm, idx_v, ones_v, acc):
        cid = lax.axis_index("c")
        sid = lax.axis_index("s")
        wid = sid * NC + cid
        base = sid * RSUB

        pltpu.sync_copy(zeros_h, acc.at[pl.ds(base, RSUB)])
        pltpu.sync_copy(ones_h, ones_v)
        pltpu.sync_copy(dst_hbm.at[wid], idx_v)
        plsc.subcore_barrier()

        @pl.loop(0, C)
        def _(j):
            pltpu.sync_copy(ones_v, acc.at[idx_v.at[j]], add=True)

        plsc.subcore_barrier()
        pltpu.sync_copy(acc.at[pl.ds(base, RSUB)],
                        out_hbm.at[cid, pl.ds(base, RSUB)])

    return deg_kernel(dst_r, ones_hbm, zeros_hbm)


def _sc_aggregate(y, src_r, dst_r, zeros_hbm):
    @functools.partial(
        pl.kernel,
        out_type=jax.ShapeDtypeStruct((NC, N_PAD, D), jnp.float32),
        mesh=_MESH,
        scratch_types=[
            pltpu.VMEM((C, K), jnp.int32),
            pltpu.VMEM((C, K), jnp.int32),
            pltpu.VMEM((K, D), jnp.float32),
            pltpu.VMEM_SHARED((N_PAD, D), jnp.float32),
        ],
    )
    def agg_kernel(y_hbm, src_hbm, dst_hbm, zeros_h, out_hbm,
                   src_v, dst_v, buf, acc):
        cid = lax.axis_index("c")
        sid = lax.axis_index("s")
        wid = sid * NC + cid
        base = sid * RSUB

        pltpu.sync_copy(zeros_h, acc.at[pl.ds(base, RSUB)])
        pltpu.sync_copy(src_hbm.at[wid], src_v)
        pltpu.sync_copy(dst_hbm.at[wid], dst_v)
        plsc.subcore_barrier()

        @pl.loop(0, C)
        def _(j):
            pltpu.sync_copy(y_hbm.at[src_v.at[j]], buf)
            pltpu.sync_copy(buf, acc.at[dst_v.at[j]], add=True)

        plsc.subcore_barrier()
        pltpu.sync_copy(acc.at[pl.ds(base, RSUB)],
                        out_hbm.at[cid, pl.ds(base, RSUB)])

    return agg_kernel(y, src_r, dst_r, zeros_hbm)


_BN = 1000


def _dot(a, b):
    return lax.dot_general(a, b, (((1,), (0,)), ((), ())),
                           precision=lax.Precision.HIGHEST,
                           preferred_element_type=jnp.float32)


def _norm_from_deg(deg_ref):
    deg = deg_ref[0] + deg_ref[1]
    return lax.rsqrt(jnp.maximum(deg, 1.0))


def _tc_matmul(x, W):
    def body(x_ref, w_ref, o_ref):
        o_ref[...] = _dot(x_ref[...], w_ref[...])

    return pl.pallas_call(
        body,
        grid=(N // _BN,),
        in_specs=[pl.BlockSpec((_BN, D), lambda i: (i, 0)),
                  pl.BlockSpec((D, D), lambda i: (0, 0))],
        out_specs=pl.BlockSpec((_BN, D), lambda i: (i, 0)),
        out_shape=jax.ShapeDtypeStruct((N, D), jnp.float32),
    )(x, W)


def _tc_scale(degp, xw):
    def body(deg_ref, xw_ref, o_ref):
        o_ref[...] = xw_ref[...] * _norm_from_deg(deg_ref)

    return pl.pallas_call(
        body,
        grid=(N // _BN,),
        in_specs=[pl.BlockSpec((NC, _BN, D), lambda i: (0, i, 0)),
                  pl.BlockSpec((_BN, D), lambda i: (i, 0))],
        out_specs=pl.BlockSpec((_BN, D), lambda i: (i, 0)),
        out_shape=jax.ShapeDtypeStruct((N, D), jnp.float32),
    )(degp, xw)


def _tc_mid(degp, sp, W2):
    def body(deg_ref, p_ref, w_ref, o_ref):
        norm = _norm_from_deg(deg_ref)
        s = p_ref[0] + p_ref[1]
        h = jnp.maximum(s * norm, 0.0)
        o_ref[...] = _dot(h * norm, w_ref[...])

    return pl.pallas_call(
        body,
        grid=(N // _BN,),
        in_specs=[pl.BlockSpec((NC, _BN, D), lambda i: (0, i, 0)),
                  pl.BlockSpec((NC, _BN, D), lambda i: (0, i, 0)),
                  pl.BlockSpec((D, D), lambda i: (0, 0))],
        out_specs=pl.BlockSpec((_BN, D), lambda i: (i, 0)),
        out_shape=jax.ShapeDtypeStruct((N, D), jnp.float32),
    )(degp, sp, W2)


def _tc_final(degp, sp):
    def body(deg_ref, p_ref, o_ref):
        o_ref[...] = (p_ref[0] + p_ref[1]) * _norm_from_deg(deg_ref)

    return pl.pallas_call(
        body,
        grid=(N // _BN,),
        in_specs=[pl.BlockSpec((NC, _BN, D), lambda i: (0, i, 0)),
                  pl.BlockSpec((NC, _BN, D), lambda i: (0, i, 0))],
        out_specs=pl.BlockSpec((_BN, D), lambda i: (i, 0)),
        out_shape=jax.ShapeDtypeStruct((N, D), jnp.float32),
    )(degp, sp)


def kernel(t, x, edge_index, W1, W2):
    src_r = edge_index[0].reshape(NW, C, K)
    dst_r = edge_index[1].reshape(NW, C, K)
    onesD = jnp.ones((K, D), jnp.float32)
    zerosD = jnp.zeros((RSUB, D), jnp.float32)

    degp = _sc_degree(dst_r, onesD, zerosD)
    xw1 = _tc_matmul(x, W1)
    y1 = _tc_scale(degp, xw1)
    s1p = _sc_aggregate(y1, src_r, dst_r, zerosD)
    y2 = _tc_mid(degp, s1p, W2)
    s2p = _sc_aggregate(y2, src_r, dst_r, zerosD)
    return _tc_final(degp, s2p)


# speedup vs baseline: 18.6569x; 1.2260x over previous
import functools

import jax
import jax.numpy as jnp
from jax import lax
from jax.experimental import pallas as pl
from jax.experimental.pallas import tpu as pltpu
from jax.experimental.pallas import tpu_sc as plsc

N = 10000
E = 320000
D = 128

NC = 2
NS = 16
NW = NC * NS
EPW = E // NW
K = 40
C = EPW // K
BC = 50
IB = C // BC
N_PAD = 10240
RSUB = N_PAD // NS
DEPTH = 5

_MESH = plsc.VectorSubcoreMesh(core_axis_name="c", subcore_axis_name="s")


def _sc_degree(dst_r, ones_hbm, zeros_hbm):
    @functools.partial(
        pl.kernel,
        out_type=jax.ShapeDtypeStruct((NC, N_PAD, D), jnp.float32),
        mesh=_MESH,
        scratch_types=[
            pltpu.VMEM((BC, K), jnp.int32),
            pltpu.VMEM((K, D), jnp.float32),
            pltpu.VMEM_SHARED((N_PAD, D), jnp.float32),
        ] + [pltpu.SemaphoreType.DMA] * DEPTH,
    )
    def deg_kernel(dst_hbm, ones_h, zeros_h, out_hbm, idx_v, ones_v, acc,
                   *sems):
        cid = lax.axis_index("c")
        sid = lax.axis_index("s")
        wid = sid * NC + cid
        base = sid * RSUB

        pltpu.sync_copy(zeros_h, acc.at[pl.ds(base, RSUB)])
        pltpu.sync_copy(ones_h, ones_v)
        plsc.subcore_barrier()

        @pl.loop(0, IB)
        def _(b):
            pltpu.sync_copy(dst_hbm.at[wid * IB + b], idx_v)

            @pl.loop(0, BC // DEPTH)
            def _(i):
                j0 = i * DEPTH
                hs = [pltpu.async_copy(ones_v, acc.at[idx_v.at[j0 + k]],
                                       sems[k], add=True)
                      for k in range(DEPTH)]
                for h in hs:
                    h.wait()

        plsc.subcore_barrier()
        pltpu.sync_copy(acc.at[pl.ds(base, RSUB)],
                        out_hbm.at[cid, pl.ds(base, RSUB)])

    return deg_kernel(dst_r, ones_hbm, zeros_hbm)


def _sc_aggregate(y, src_r, dst_r, zeros_hbm):
    @functools.partial(
        pl.kernel,
        out_type=jax.ShapeDtypeStruct((NC, N_PAD, D), jnp.float32),
        mesh=_MESH,
        scratch_types=[
            pltpu.VMEM((BC, K), jnp.int32),
            pltpu.VMEM((BC, K), jnp.int32),
        ] + [pltpu.VMEM((K, D), jnp.float32)] * DEPTH
          + [pltpu.VMEM_SHARED((N_PAD, D), jnp.float32)]
          + [pltpu.SemaphoreType.DMA] * (2 * DEPTH),
    )
    def agg_kernel(y_hbm, src_hbm, dst_hbm, zeros_h, out_hbm,
                   src_v, dst_v, *rest):
        bufs = rest[:DEPTH]
        acc = rest[DEPTH]
        gsems = rest[DEPTH + 1:2 * DEPTH + 1]
        ssems = rest[2 * DEPTH + 1:]
        cid = lax.axis_index("c")
        sid = lax.axis_index("s")
        wid = sid * NC + cid
        base = sid * RSUB

        pltpu.sync_copy(zeros_h, acc.at[pl.ds(base, RSUB)])
        plsc.subcore_barrier()

        @pl.loop(0, IB)
        def _(b):
            pltpu.sync_copy(src_hbm.at[wid * IB + b], src_v)
            pltpu.sync_copy(dst_hbm.at[wid * IB + b], dst_v)

            @pl.loop(0, BC // DEPTH)
            def _(i):
                j0 = i * DEPTH
                ghs = [pltpu.async_copy(y_hbm.at[src_v.at[j0 + k]],
                                        bufs[k], gsems[k])
                       for k in range(DEPTH)]
                shs = []
                for k in range(DEPTH):
                    ghs[k].wait()
                    shs.append(pltpu.async_copy(bufs[k],
                                                acc.at[dst_v.at[j0 + k]],
                                                ssems[k], add=True))
                for h in shs:
                    h.wait()

        plsc.subcore_barrier()
        pltpu.sync_copy(acc.at[pl.ds(base, RSUB)],
                        out_hbm.at[cid, pl.ds(base, RSUB)])

    return agg_kernel(y, src_r, dst_r, zeros_hbm)


_BN = 1000


def _dot(a, b):
    return lax.dot_general(a, b, (((1,), (0,)), ((), ())),
                           precision=lax.Precision.HIGHEST,
                           preferred_element_type=jnp.float32)


def _norm_from_deg(deg_ref):
    deg = deg_ref[0] + deg_ref[1]
    return lax.rsqrt(jnp.maximum(deg, 1.0))


def _tc_matmul(x, W):
    def body(x_ref, w_ref, o_ref):
        o_ref[...] = _dot(x_ref[...], w_ref[...])

    return pl.pallas_call(
        body,
        grid=(N // _BN,),
        in_specs=[pl.BlockSpec((_BN, D), lambda i: (i, 0)),
                  pl.BlockSpec((D, D), lambda i: (0, 0))],
        out_specs=pl.BlockSpec((_BN, D), lambda i: (i, 0)),
        out_shape=jax.ShapeDtypeStruct((N, D), jnp.float32),
    )(x, W)


def _tc_scale(degp, xw):
    def body(deg_ref, xw_ref, o_ref):
        o_ref[...] = xw_ref[...] * _norm_from_deg(deg_ref)

    return pl.pallas_call(
        body,
        grid=(N // _BN,),
        in_specs=[pl.BlockSpec((NC, _BN, D), lambda i: (0, i, 0)),
                  pl.BlockSpec((_BN, D), lambda i: (i, 0))],
        out_specs=pl.BlockSpec((_BN, D), lambda i: (i, 0)),
        out_shape=jax.ShapeDtypeStruct((N, D), jnp.float32),
    )(degp, xw)


def _tc_mid(degp, sp, W2):
    def body(deg_ref, p_ref, w_ref, o_ref):
        norm = _norm_from_deg(deg_ref)
        s = p_ref[0] + p_ref[1]
        h = jnp.maximum(s * norm, 0.0)
        o_ref[...] = _dot(h * norm, w_ref[...])

    return pl.pallas_call(
        body,
        grid=(N // _BN,),
        in_specs=[pl.BlockSpec((NC, _BN, D), lambda i: (0, i, 0)),
                  pl.BlockSpec((NC, _BN, D), lambda i: (0, i, 0)),
                  pl.BlockSpec((D, D), lambda i: (0, 0))],
        out_specs=pl.BlockSpec((_BN, D), lambda i: (i, 0)),
        out_shape=jax.ShapeDtypeStruct((N, D), jnp.float32),
    )(degp, sp, W2)


def _tc_final(degp, sp):
    def body(deg_ref, p_ref, o_ref):
        o_ref[...] = (p_ref[0] + p_ref[1]) * _norm_from_deg(deg_ref)

    return pl.pallas_call(
        body,
        grid=(N // _BN,),
        in_specs=[pl.BlockSpec((NC, _BN, D), lambda i: (0, i, 0)),
                  pl.BlockSpec((NC, _BN, D), lambda i: (0, i, 0))],
        out_specs=pl.BlockSpec((_BN, D), lambda i: (i, 0)),
        out_shape=jax.ShapeDtypeStruct((N, D), jnp.float32),
    )(degp, sp)


def kernel(t, x, edge_index, W1, W2):
    src_r = edge_index[0].reshape(NW * IB, BC, K)
    dst_r = edge_index[1].reshape(NW * IB, BC, K)
    onesD = jnp.ones((K, D), jnp.float32)
    zerosD = jnp.zeros((RSUB, D), jnp.float32)

    degp = _sc_degree(dst_r, onesD, zerosD)
    xw1 = _tc_matmul(x, W1)
    y1 = _tc_scale(degp, xw1)
    s1p = _sc_aggregate(y1, src_r, dst_r, zerosD)
    y2 = _tc_mid(degp, s1p, W2)
    s2p = _sc_aggregate(y2, src_r, dst_r, zerosD)
    return _tc_final(degp, s2p)


# speedup vs baseline: 18.9521x; 1.0158x over previous
import functools

import jax
import jax.numpy as jnp
from jax import lax
from jax.experimental import pallas as pl
from jax.experimental.pallas import tpu as pltpu
from jax.experimental.pallas import tpu_sc as plsc

N = 10000
E = 320000
D = 128

NC = 2
NS = 16
NW = NC * NS
EPW = E // NW
K = 40
C = EPW // K
BC = 50
IB = C // BC
KA = 16
CA = EPW // KA
N_PAD = 10240
RSUB = N_PAD // NS
DEPTH = 5

_MESH = plsc.VectorSubcoreMesh(core_axis_name="c", subcore_axis_name="s")


def _sc_degree(dst_r, ones_hbm, zeros_hbm):
    @functools.partial(
        pl.kernel,
        out_type=jax.ShapeDtypeStruct((NC, N_PAD, D), jnp.float32),
        mesh=_MESH,
        scratch_types=[
            pltpu.VMEM((BC, K), jnp.int32),
            pltpu.VMEM((K, D), jnp.float32),
            pltpu.VMEM_SHARED((N_PAD, D), jnp.float32),
        ] + [pltpu.SemaphoreType.DMA] * DEPTH,
    )
    def deg_kernel(dst_hbm, ones_h, zeros_h, out_hbm, idx_v, ones_v, acc,
                   *sems):
        cid = lax.axis_index("c")
        sid = lax.axis_index("s")
        wid = sid * NC + cid
        base = sid * RSUB

        pltpu.sync_copy(zeros_h, acc.at[pl.ds(base, RSUB)])
        pltpu.sync_copy(ones_h, ones_v)
        plsc.subcore_barrier()

        @pl.loop(0, IB)
        def _(b):
            pltpu.sync_copy(dst_hbm.at[wid * IB + b], idx_v)

            for k in range(DEPTH):
                pltpu.async_copy(ones_v, acc.at[idx_v.at[k]],
                                 sems[k], add=True)

            @pl.loop(1, BC // DEPTH)
            def _(i):
                j0 = i * DEPTH
                for k in range(DEPTH):
                    pltpu.make_async_copy(ones_v, acc.at[idx_v.at[j0 + k]],
                                          sems[k]).wait()
                    pltpu.async_copy(ones_v, acc.at[idx_v.at[j0 + k]],
                                     sems[k], add=True)

            for k in range(DEPTH):
                pltpu.make_async_copy(ones_v, acc.at[idx_v.at[k]],
                                      sems[k]).wait()

        plsc.subcore_barrier()
        pltpu.sync_copy(acc.at[pl.ds(base, RSUB)],
                        out_hbm.at[cid, pl.ds(base, RSUB)])

    return deg_kernel(dst_r, ones_hbm, zeros_hbm)


def _sc_aggregate(y, src_r, dst_r, zeros_hbm):
    @functools.partial(
        pl.kernel,
        out_type=jax.ShapeDtypeStruct((NC, N_PAD, D), jnp.float32),
        mesh=_MESH,
        scratch_types=[
            pltpu.VMEM((EPW,), jnp.int32),
            pltpu.VMEM((EPW,), jnp.int32),
        ] + [pltpu.VMEM((KA, D), jnp.float32)] * DEPTH
          + [pltpu.VMEM_SHARED((N_PAD, D), jnp.float32)]
          + [pltpu.SemaphoreType.DMA] * (2 * DEPTH),
    )
    def agg_kernel(y_hbm, src_hbm, dst_hbm, zeros_h, out_hbm,
                   src_v, dst_v, *rest):
        bufs = rest[:DEPTH]
        acc = rest[DEPTH]
        gsems = rest[DEPTH + 1:2 * DEPTH + 1]
        ssems = rest[2 * DEPTH + 1:]
        cid = lax.axis_index("c")
        sid = lax.axis_index("s")
        wid = sid * NC + cid
        base = sid * RSUB

        pltpu.sync_copy(zeros_h, acc.at[pl.ds(base, RSUB)])
        pltpu.sync_copy(src_hbm.at[wid], src_v)
        pltpu.sync_copy(dst_hbm.at[wid], dst_v)
        plsc.subcore_barrier()

        def gather(j, k):
            pltpu.async_copy(y_hbm.at[src_v[pl.ds(j * KA, KA)]], bufs[k],
                             gsems[k])

        def scatter(j, k):
            pltpu.async_copy(bufs[k], acc.at[dst_v[pl.ds(j * KA, KA)]],
                             ssems[k], add=True)

        def wait_gather(k):
            pltpu.make_async_copy(y_hbm.at[src_v[pl.ds(0, KA)]], bufs[k],
                                  gsems[k]).wait()

        def wait_scatter(k):
            pltpu.make_async_copy(bufs[k], acc.at[dst_v[pl.ds(0, KA)]],
                                  ssems[k]).wait()

        for k in range(DEPTH):
            gather(k, k)

        @pl.loop(1, CA // DEPTH)
        def _(i):
            j0 = i * DEPTH
            for k in range(DEPTH):
                wait_gather(k)
                scatter(j0 - DEPTH + k, k)
            for k in range(DEPTH):
                wait_scatter(k)
                gather(j0 + k, k)

        for k in range(DEPTH):
            wait_gather(k)
            scatter(CA - DEPTH + k, k)
        for k in range(DEPTH):
            wait_scatter(k)

        plsc.subcore_barrier()
        pltpu.sync_copy(acc.at[pl.ds(base, RSUB)],
                        out_hbm.at[cid, pl.ds(base, RSUB)])

    return agg_kernel(y, src_r, dst_r, zeros_hbm)


_BN = 1000


def _dot(a, b):
    return lax.dot_general(a, b, (((1,), (0,)), ((), ())),
                           precision=lax.Precision.HIGHEST,
                           preferred_element_type=jnp.float32)


def _norm_from_deg(deg_ref):
    deg = deg_ref[0] + deg_ref[1]
    return lax.rsqrt(jnp.maximum(deg, 1.0))


def _tc_matmul(x, W):
    def body(x_ref, w_ref, o_ref):
        o_ref[...] = _dot(x_ref[...], w_ref[...])

    return pl.pallas_call(
        body,
        grid=(N // _BN,),
        in_specs=[pl.BlockSpec((_BN, D), lambda i: (i, 0)),
                  pl.BlockSpec((D, D), lambda i: (0, 0))],
        out_specs=pl.BlockSpec((_BN, D), lambda i: (i, 0)),
        out_shape=jax.ShapeDtypeStruct((N, D), jnp.float32),
    )(x, W)


def _tc_scale(degp, xw):
    def body(deg_ref, xw_ref, o_ref):
        o_ref[...] = xw_ref[...] * _norm_from_deg(deg_ref)

    return pl.pallas_call(
        body,
        grid=(N // _BN,),
        in_specs=[pl.BlockSpec((NC, _BN, D), lambda i: (0, i, 0)),
                  pl.BlockSpec((_BN, D), lambda i: (i, 0))],
        out_specs=pl.BlockSpec((_BN, D), lambda i: (i, 0)),
        out_shape=jax.ShapeDtypeStruct((N, D), jnp.float32),
    )(degp, xw)


def _tc_mid(degp, sp, W2):
    def body(deg_ref, p_ref, w_ref, o_ref):
        norm = _norm_from_deg(deg_ref)
        s = p_ref[0] + p_ref[1]
        h = jnp.maximum(s * norm, 0.0)
        o_ref[...] = _dot(h * norm, w_ref[...])

    return pl.pallas_call(
        body,
        grid=(N // _BN,),
        in_specs=[pl.BlockSpec((NC, _BN, D), lambda i: (0, i, 0)),
                  pl.BlockSpec((NC, _BN, D), lambda i: (0, i, 0)),
                  pl.BlockSpec((D, D), lambda i: (0, 0))],
        out_specs=pl.BlockSpec((_BN, D), lambda i: (i, 0)),
        out_shape=jax.ShapeDtypeStruct((N, D), jnp.float32),
    )(degp, sp, W2)


def _tc_final(degp, sp):
    def body(deg_ref, p_ref, o_ref):
        o_ref[...] = (p_ref[0] + p_ref[1]) * _norm_from_deg(deg_ref)

    return pl.pallas_call(
        body,
        grid=(N // _BN,),
        in_specs=[pl.BlockSpec((NC, _BN, D), lambda i: (0, i, 0)),
                  pl.BlockSpec((NC, _BN, D), lambda i: (0, i, 0))],
        out_specs=pl.BlockSpec((_BN, D), lambda i: (i, 0)),
        out_shape=jax.ShapeDtypeStruct((N, D), jnp.float32),
    )(degp, sp)


def kernel(t, x, edge_index, W1, W2):
    src_r = edge_index[0].reshape(NW, EPW)
    dst_r = edge_index[1].reshape(NW, EPW)
    dst_deg = edge_index[1].reshape(NW * IB, BC, K)
    onesD = jnp.ones((K, D), jnp.float32)
    zerosD = jnp.zeros((RSUB, D), jnp.float32)

    degp = _sc_degree(dst_deg, onesD, zerosD)
    xw1 = _tc_matmul(x, W1)
    y1 = _tc_scale(degp, xw1)
    s1p = _sc_aggregate(y1, src_r, dst_r, zerosD)
    y2 = _tc_mid(degp, s1p, W2)
    s2p = _sc_aggregate(y2, src_r, dst_r, zerosD)
    return _tc_final(degp, s2p)


# speedup vs baseline: 21.5752x; 1.1384x over previous
import dataclasses
import functools

import jax
import jax.numpy as jnp
from jax import lax
from jax.experimental import pallas as pl
from jax.experimental.pallas import tpu as pltpu
from jax.experimental.pallas import tpu_sc as plsc

N = 10000
E = 320000
D = 128

NC = 2
NS = 16
NW = NC * NS
EPW = E // NW
KA = 16
CA = EPW // KA
N_PAD = 10240
RSUB = N_PAD // NS
DEPTH = 5
HR = N_PAD // D

_MESH = plsc.VectorSubcoreMesh(core_axis_name="c", subcore_axis_name="s")

_CP = pltpu.CompilerParams()
if "needs_layout_passes" in pltpu.CompilerParams.__dataclass_fields__:
    _CP = dataclasses.replace(_CP, needs_layout_passes=False)


def _sc_degree(dst_r, zeros_hbm):
    @functools.partial(
        pl.kernel,
        out_type=jax.ShapeDtypeStruct((NC, NS, HR, D), jnp.float32),
        mesh=_MESH,
        compiler_params=_CP,
        scratch_types=[
            pltpu.VMEM((EPW,), jnp.int32),
            pltpu.VMEM((HR, D), jnp.float32),
        ],
    )
    def deg_kernel(dst_hbm, zeros_h, out_hbm, idx_v, hist_v):
        cid = lax.axis_index("c")
        sid = lax.axis_index("s")
        wid = sid * NC + cid

        pltpu.sync_copy(zeros_h.at[pl.ds(0, HR)], hist_v)
        pltpu.sync_copy(dst_hbm.at[wid], idx_v)

        @pl.loop(0, EPW // 16)
        def _(j):
            iv = idx_v[pl.ds(j * 16, 16)]
            counts, last = plsc.scan_count(iv)
            plsc.addupdate_scatter(
                hist_v,
                [lax.shift_right_logical(iv, 7), lax.bitwise_and(iv, 127)],
                counts.astype(jnp.float32),
                mask=last,
            )

        pltpu.sync_copy(hist_v, out_hbm.at[cid, sid])

    return deg_kernel(dst_r, zeros_hbm)


def _sc_aggregate(y, src_r, dst_r, zeros_hbm):
    @functools.partial(
        pl.kernel,
        out_type=jax.ShapeDtypeStruct((NC, N_PAD, D), jnp.float32),
        mesh=_MESH,
        scratch_types=[
            pltpu.VMEM((EPW,), jnp.int32),
            pltpu.VMEM((EPW,), jnp.int32),
        ] + [pltpu.VMEM((KA, D), jnp.float32)] * DEPTH
          + [pltpu.VMEM_SHARED((N_PAD, D), jnp.float32)]
          + [pltpu.SemaphoreType.DMA] * (2 * DEPTH),
    )
    def agg_kernel(y_hbm, src_hbm, dst_hbm, zeros_h, out_hbm,
                   src_v, dst_v, *rest):
        bufs = rest[:DEPTH]
        acc = rest[DEPTH]
        gsems = rest[DEPTH + 1:2 * DEPTH + 1]
        ssems = rest[2 * DEPTH + 1:]
        cid = lax.axis_index("c")
        sid = lax.axis_index("s")
        wid = sid * NC + cid
        base = sid * RSUB

        pltpu.sync_copy(zeros_h, acc.at[pl.ds(base, RSUB)])
        pltpu.sync_copy(src_hbm.at[wid], src_v)
        pltpu.sync_copy(dst_hbm.at[wid], dst_v)
        plsc.subcore_barrier()

        def gather(j, k):
            pltpu.async_copy(y_hbm.at[src_v[pl.ds(j * KA, KA)]], bufs[k],
                             gsems[k])

        def scatter(j, k):
            pltpu.async_copy(bufs[k], acc.at[dst_v[pl.ds(j * KA, KA)]],
                             ssems[k], add=True)

        def wait_gather(k):
            pltpu.make_async_copy(y_hbm.at[src_v[pl.ds(0, KA)]], bufs[k],
                                  gsems[k]).wait()

        def wait_scatter(k):
            pltpu.make_async_copy(bufs[k], acc.at[dst_v[pl.ds(0, KA)]],
                                  ssems[k]).wait()

        for k in range(DEPTH):
            gather(k, k)

        @pl.loop(1, CA // DEPTH)
        def _(i):
            j0 = i * DEPTH
            for k in range(DEPTH):
                wait_gather(k)
                scatter(j0 - DEPTH + k, k)
            for k in range(DEPTH):
                wait_scatter(k)
                gather(j0 + k, k)

        for k in range(DEPTH):
            wait_gather(k)
            scatter(CA - DEPTH + k, k)
        for k in range(DEPTH):
            wait_scatter(k)

        plsc.subcore_barrier()
        pltpu.sync_copy(acc.at[pl.ds(base, RSUB)],
                        out_hbm.at[cid, pl.ds(base, RSUB)])

    return agg_kernel(y, src_r, dst_r, zeros_hbm)


_BN = 1024
_GRID = (N_PAD // _BN,)


def _dot(a, b):
    return lax.dot_general(a, b, (((1,), (0,)), ((), ())),
                           precision=lax.Precision.HIGHEST,
                           preferred_element_type=jnp.float32)


def _tc_prep(hists):
    def body(h_ref, o_ref):
        deg = jnp.sum(h_ref[...], axis=(0, 1))
        norm = lax.rsqrt(jnp.maximum(deg, 1.0))
        ones = jnp.ones((1, D), jnp.float32)
        for a in range(8):
            o_ref[pl.ds(a * D, D), :] = lax.dot_general(
                norm[a:a + 1, :], ones, (((0,), (0,)), ((), ())),
                precision=lax.Precision.HIGHEST,
                preferred_element_type=jnp.float32)

    return pl.pallas_call(
        body,
        grid=(HR // 8,),
        in_specs=[pl.BlockSpec((NC, NS, 8, D), lambda r: (0, 0, r, 0))],
        out_specs=pl.BlockSpec((8 * D, D), lambda r: (r, 0)),
        out_shape=jax.ShapeDtypeStruct((N_PAD, D), jnp.float32),
    )(hists)


def _tc_first(normb, x, W1):
    def body(n_ref, x_ref, w_ref, o_ref):
        o_ref[...] = _dot(x_ref[...] * n_ref[...], w_ref[...])

    return pl.pallas_call(
        body,
        grid=_GRID,
        in_specs=[pl.BlockSpec((_BN, D), lambda i: (i, 0)),
                  pl.BlockSpec((_BN, D), lambda i: (i, 0)),
                  pl.BlockSpec((D, D), lambda i: (0, 0))],
        out_specs=pl.BlockSpec((_BN, D), lambda i: (i, 0)),
        out_shape=jax.ShapeDtypeStruct((N, D), jnp.float32),
    )(normb, x, W1)


def _tc_mid(normb, sp, W2):
    def body(n_ref, p_ref, w_ref, o_ref):
        norm = n_ref[...]
        h = jnp.maximum((p_ref[0] + p_ref[1]) * norm, 0.0)
        o_ref[...] = _dot(h * norm, w_ref[...])

    return pl.pallas_call(
        body,
        grid=_GRID,
        in_specs=[pl.BlockSpec((_BN, D), lambda i: (i, 0)),
                  pl.BlockSpec((NC, _BN, D), lambda i: (0, i, 0)),
                  pl.BlockSpec((D, D), lambda i: (0, 0))],
        out_specs=pl.BlockSpec((_BN, D), lambda i: (i, 0)),
        out_shape=jax.ShapeDtypeStruct((N, D), jnp.float32),
    )(normb, sp, W2)


def _tc_final(normb, sp):
    def body(n_ref, p_ref, o_ref):
        o_ref[...] = (p_ref[0] + p_ref[1]) * n_ref[...]

    return pl.pallas_call(
        body,
        grid=_GRID,
        in_specs=[pl.BlockSpec((_BN, D), lambda i: (i, 0)),
                  pl.BlockSpec((NC, _BN, D), lambda i: (0, i, 0))],
        out_specs=pl.BlockSpec((_BN, D), lambda i: (i, 0)),
        out_shape=jax.ShapeDtypeStruct((N, D), jnp.float32),
    )(normb, sp)


def kernel(t, x, edge_index, W1, W2):
    src_r = edge_index[0].reshape(NW, EPW)
    dst_r = edge_index[1].reshape(NW, EPW)
    zerosD = jnp.zeros((RSUB, D), jnp.float32)

    hists = _sc_degree(dst_r, zerosD)
    normb = _tc_prep(hists)
    y1 = _tc_first(normb, x, W1)
    s1p = _sc_aggregate(y1, src_r, dst_r, zerosD)
    y2 = _tc_mid(normb, s1p, W2)
    s2p = _sc_aggregate(y2, src_r, dst_r, zerosD)
    return _tc_final(normb, s2p)


# speedup vs baseline: 22.0113x; 1.0202x over previous
import dataclasses
import functools

import jax
import jax.numpy as jnp
from jax import lax
from jax.experimental import pallas as pl
from jax.experimental.pallas import tpu as pltpu
from jax.experimental.pallas import tpu_sc as plsc

N = 10000
E = 320000
D = 128

NC = 2
NS = 16
NW = NC * NS
EPW = E // NW
KA = 16
CA = EPW // KA
N_PAD = 10240
RSUB = N_PAD // NS
DEPTH = 5
HR = N_PAD // D

_MESH = plsc.VectorSubcoreMesh(core_axis_name="c", subcore_axis_name="s")

_CP = pltpu.CompilerParams()
if "needs_layout_passes" in pltpu.CompilerParams.__dataclass_fields__:
    _CP = dataclasses.replace(_CP, needs_layout_passes=False)


def _sc_degree(dst_r, zeros_hbm):
    @functools.partial(
        pl.kernel,
        out_type=jax.ShapeDtypeStruct((NC, NS, HR, D), jnp.float32),
        mesh=_MESH,
        compiler_params=_CP,
        scratch_types=[
            pltpu.VMEM((EPW,), jnp.int32),
            pltpu.VMEM((HR, D), jnp.float32),
        ],
    )
    def deg_kernel(dst_hbm, zeros_h, out_hbm, idx_v, hist_v):
        cid = lax.axis_index("c")
        sid = lax.axis_index("s")
        wid = sid * NC + cid

        pltpu.sync_copy(zeros_h.at[pl.ds(0, HR)], hist_v)
        pltpu.sync_copy(dst_hbm.at[wid], idx_v)

        @pl.loop(0, EPW // 16)
        def _(j):
            iv = idx_v[pl.ds(j * 16, 16)]
            counts, last = plsc.scan_count(iv)
            plsc.addupdate_scatter(
                hist_v,
                [lax.shift_right_logical(iv, 7), lax.bitwise_and(iv, 127)],
                counts.astype(jnp.float32),
                mask=last,
            )

        pltpu.sync_copy(hist_v, out_hbm.at[cid, sid])

    return deg_kernel(dst_r, zeros_hbm)


def _sc_aggregate(y, src_r, dst_r, zeros_hbm):
    @functools.partial(
        pl.kernel,
        out_type=jax.ShapeDtypeStruct((NC, N_PAD, D), jnp.float32),
        mesh=_MESH,
        scratch_types=[
            pltpu.VMEM((EPW,), jnp.int32),
            pltpu.VMEM((EPW,), jnp.int32),
        ] + [pltpu.VMEM((KA, D), jnp.float32)] * DEPTH
          + [pltpu.VMEM_SHARED((N_PAD, D), jnp.float32)]
          + [pltpu.SemaphoreType.DMA] * (2 * DEPTH),
    )
    def agg_kernel(y_hbm, src_hbm, dst_hbm, zeros_h, out_hbm,
                   src_v, dst_v, *rest):
        bufs = rest[:DEPTH]
        acc = rest[DEPTH]
        gsems = rest[DEPTH + 1:2 * DEPTH + 1]
        ssems = rest[2 * DEPTH + 1:]
        cid = lax.axis_index("c")
        sid = lax.axis_index("s")
        wid = sid * NC + cid
        base = sid * RSUB

        pltpu.sync_copy(zeros_h, acc.at[pl.ds(base, RSUB)])
        pltpu.sync_copy(src_hbm.at[wid], src_v)
        pltpu.sync_copy(dst_hbm.at[wid], dst_v)
        plsc.subcore_barrier()

        def gather(j, k):
            pltpu.async_copy(y_hbm.at[src_v[pl.ds(j * KA, KA)]], bufs[k],
                             gsems[k])

        def scatter(j, k):
            pltpu.async_copy(bufs[k], acc.at[dst_v[pl.ds(j * KA, KA)]],
                             ssems[k], add=True)

        def wait_gather(k):
            pltpu.make_async_copy(y_hbm.at[src_v[pl.ds(0, KA)]], bufs[k],
                                  gsems[k]).wait()

        def wait_scatter(k):
            pltpu.make_async_copy(bufs[k], acc.at[dst_v[pl.ds(0, KA)]],
                                  ssems[k]).wait()

        for k in range(DEPTH):
            gather(k, k)

        @pl.loop(1, CA // DEPTH)
        def _(i):
            j0 = i * DEPTH
            for k in range(DEPTH):
                wait_gather(k)
                scatter(j0 - DEPTH + k, k)
            for k in range(DEPTH):
                wait_scatter(k)
                gather(j0 + k, k)

        for k in range(DEPTH):
            wait_gather(k)
            scatter(CA - DEPTH + k, k)
        for k in range(DEPTH):
            wait_scatter(k)

        plsc.subcore_barrier()
        pltpu.sync_copy(acc.at[pl.ds(base, RSUB)],
                        out_hbm.at[cid, pl.ds(base, RSUB)])

    return agg_kernel(y, src_r, dst_r, zeros_hbm)


_BN = 1024
_GRID = (N_PAD // _BN,)


def _dot(a, b):
    return lax.dot_general(a, b, (((1,), (0,)), ((), ())),
                           precision=lax.Precision.HIGHEST,
                           preferred_element_type=jnp.float32)


def _tc_mm1(x, W1):
    def body(x_ref, w_ref, o_ref):
        o_ref[...] = _dot(x_ref[...], w_ref[...])

    return pl.pallas_call(
        body,
        grid=_GRID,
        in_specs=[pl.BlockSpec((_BN, D), lambda i: (i, 0)),
                  pl.BlockSpec((D, D), lambda i: (0, 0))],
        out_specs=pl.BlockSpec((_BN, D), lambda i: (i, 0)),
        out_shape=jax.ShapeDtypeStruct((N, D), jnp.float32),
    )(x, W1)


def _tc_prep(hists, xw1):
    def body(h_ref, xw_ref, n_ref, y_ref):
        deg = jnp.sum(h_ref[...], axis=(0, 1))
        norm = lax.rsqrt(jnp.maximum(deg, 1.0))
        ones = jnp.ones((1, D), jnp.float32)
        for a in range(8):
            n_ref[pl.ds(a * D, D), :] = lax.dot_general(
                norm[a:a + 1, :], ones, (((0,), (0,)), ((), ())),
                precision=lax.Precision.HIGHEST,
                preferred_element_type=jnp.float32)
        y_ref[...] = xw_ref[...] * n_ref[...]

    return pl.pallas_call(
        body,
        grid=(HR // 8,),
        in_specs=[pl.BlockSpec((NC, NS, 8, D), lambda r: (0, 0, r, 0)),
                  pl.BlockSpec((8 * D, D), lambda r: (r, 0))],
        out_specs=[pl.BlockSpec((8 * D, D), lambda r: (r, 0)),
                   pl.BlockSpec((8 * D, D), lambda r: (r, 0))],
        out_shape=[jax.ShapeDtypeStruct((N_PAD, D), jnp.float32),
                   jax.ShapeDtypeStruct((N, D), jnp.float32)],
    )(hists, xw1)


def _tc_mid(normb, sp, W2):
    def body(n_ref, p_ref, w_ref, o_ref):
        norm = n_ref[...]
        h = jnp.maximum((p_ref[0] + p_ref[1]) * norm, 0.0)
        o_ref[...] = _dot(h * norm, w_ref[...])

    return pl.pallas_call(
        body,
        grid=_GRID,
        in_specs=[pl.BlockSpec((_BN, D), lambda i: (i, 0)),
                  pl.BlockSpec((NC, _BN, D), lambda i: (0, i, 0)),
                  pl.BlockSpec((D, D), lambda i: (0, 0))],
        out_specs=pl.BlockSpec((_BN, D), lambda i: (i, 0)),
        out_shape=jax.ShapeDtypeStruct((N, D), jnp.float32),
    )(normb, sp, W2)


def _tc_final(normb, sp):
    def body(n_ref, p_ref, o_ref):
        o_ref[...] = (p_ref[0] + p_ref[1]) * n_ref[...]

    return pl.pallas_call(
        body,
        grid=_GRID,
        in_specs=[pl.BlockSpec((_BN, D), lambda i: (i, 0)),
                  pl.BlockSpec((NC, _BN, D), lambda i: (0, i, 0))],
        out_specs=pl.BlockSpec((_BN, D), lambda i: (i, 0)),
        out_shape=jax.ShapeDtypeStruct((N, D), jnp.float32),
    )(normb, sp)


def kernel(t, x, edge_index, W1, W2):
    src_r = edge_index[0].reshape(NW, EPW)
    dst_r = edge_index[1].reshape(NW, EPW)
    zerosD = jnp.zeros((RSUB, D), jnp.float32)

    hists = _sc_degree(dst_r, zerosD)
    xw1 = _tc_mm1(x, W1)
    normb, y1 = _tc_prep(hists, xw1)
    s1p = _sc_aggregate(y1, src_r, dst_r, zerosD)
    y2 = _tc_mid(normb, s1p, W2)
    s2p = _sc_aggregate(y2, src_r, dst_r, zerosD)
    return _tc_final(normb, s2p)


# speedup vs baseline: 22.3385x; 1.0149x over previous
import dataclasses
import functools

import jax
import jax.numpy as jnp
from jax import lax
from jax.experimental import pallas as pl
from jax.experimental.pallas import tpu as pltpu
from jax.experimental.pallas import tpu_sc as plsc

N = 10000
E = 320000
D = 128

NC = 2
NS = 16
NW = NC * NS
EPW = E // NW
KA = 16
CA = EPW // KA
N_PAD = 10240
RSUB = N_PAD // NS
DEPTH = 5
HR = N_PAD // D

_MESH = plsc.VectorSubcoreMesh(core_axis_name="c", subcore_axis_name="s")

_CP = pltpu.CompilerParams()
if "needs_layout_passes" in pltpu.CompilerParams.__dataclass_fields__:
    _CP = dataclasses.replace(_CP, needs_layout_passes=False)


def _sc_degree(dst_r, zeros_hbm):
    @functools.partial(
        pl.kernel,
        out_type=jax.ShapeDtypeStruct((NC, NS, HR, D), jnp.float32),
        mesh=_MESH,
        compiler_params=_CP,
        scratch_types=[
            pltpu.VMEM((EPW,), jnp.int32),
            pltpu.VMEM((HR, D), jnp.float32),
        ],
    )
    def deg_kernel(dst_hbm, zeros_h, out_hbm, idx_v, hist_v):
        cid = lax.axis_index("c")
        sid = lax.axis_index("s")
        wid = sid * NC + cid

        pltpu.sync_copy(zeros_h.at[pl.ds(0, HR)], hist_v)
        pltpu.sync_copy(dst_hbm.at[wid], idx_v)

        @pl.loop(0, EPW // 16)
        def _(j):
            iv = idx_v[pl.ds(j * 16, 16)]
            counts, last = plsc.scan_count(iv)
            plsc.addupdate_scatter(
                hist_v,
                [lax.shift_right_logical(iv, 7), lax.bitwise_and(iv, 127)],
                counts.astype(jnp.float32),
                mask=last,
            )

        pltpu.sync_copy(hist_v, out_hbm.at[cid, sid])

    return deg_kernel(dst_r, zeros_hbm)


def _sc_aggregate(y, src_r, dst_r, zeros_hbm):
    @functools.partial(
        pl.kernel,
        out_type=jax.ShapeDtypeStruct((NC, N_PAD, D), jnp.float32),
        mesh=_MESH,
        scratch_types=[
            pltpu.VMEM((EPW,), jnp.int32),
            pltpu.VMEM((EPW,), jnp.int32),
        ] + [pltpu.VMEM((KA, D), jnp.float32)] * (2 * DEPTH)
          + [pltpu.VMEM_SHARED((N_PAD, D), jnp.float32)]
          + [pltpu.SemaphoreType.DMA] * (4 * DEPTH),
    )
    def agg_kernel(y_hbm, src_hbm, dst_hbm, zeros_h, out_hbm,
                   src_v, dst_v, *rest):
        bufA = rest[:DEPTH]
        bufB = rest[DEPTH:2 * DEPTH]
        acc = rest[2 * DEPTH]
        r = 2 * DEPTH + 1
        gsemA = rest[r:r + DEPTH]
        gsemB = rest[r + DEPTH:r + 2 * DEPTH]
        ssemA = rest[r + 2 * DEPTH:r + 3 * DEPTH]
        ssemB = rest[r + 3 * DEPTH:]
        cid = lax.axis_index("c")
        sid = lax.axis_index("s")
        wid = sid * NC + cid
        base = sid * RSUB

        pltpu.sync_copy(zeros_h, acc.at[pl.ds(base, RSUB)])
        pltpu.sync_copy(src_hbm.at[wid], src_v)
        pltpu.sync_copy(dst_hbm.at[wid], dst_v)
        plsc.subcore_barrier()

        def gather(b, k, bufs, sems):
            pltpu.async_copy(
                y_hbm.at[src_v[pl.ds((b * DEPTH + k) * KA, KA)]],
                bufs[k], sems[k])

        def scatter(b, k, bufs, sems):
            pltpu.async_copy(
                bufs[k], acc.at[dst_v[pl.ds((b * DEPTH + k) * KA, KA)]],
                sems[k], add=True)

        def wait_gather(k, bufs, sems):
            pltpu.make_async_copy(y_hbm.at[src_v[pl.ds(0, KA)]], bufs[k],
                                  sems[k]).wait()

        def wait_scatter(k, bufs, sems):
            pltpu.make_async_copy(bufs[k], acc.at[dst_v[pl.ds(0, KA)]],
                                  sems[k]).wait()

        NB = CA // DEPTH

        for k in range(DEPTH):
            gather(0, k, bufA, gsemA)
        for k in range(DEPTH):
            wait_gather(k, bufA, gsemA)
            scatter(0, k, bufA, ssemA)
        for k in range(DEPTH):
            gather(1, k, bufB, gsemB)
        for k in range(DEPTH):
            wait_gather(k, bufB, gsemB)
            scatter(1, k, bufB, ssemB)
        for k in range(DEPTH):
            wait_scatter(k, bufA, ssemA)
            gather(2, k, bufA, gsemA)

        @pl.loop(1, (NB - 1) // 2)
        def _(j):
            b0 = 2 * j
            for k in range(DEPTH):
                wait_gather(k, bufA, gsemA)
                scatter(b0, k, bufA, ssemA)
            for k in range(DEPTH):
                wait_scatter(k, bufB, ssemB)
                gather(b0 + 1, k, bufB, gsemB)
            for k in range(DEPTH):
                wait_gather(k, bufB, gsemB)
                scatter(b0 + 1, k, bufB, ssemB)
            for k in range(DEPTH):
                wait_scatter(k, bufA, ssemA)
                gather(b0 + 2, k, bufA, gsemA)

        for k in range(DEPTH):
            wait_gather(k, bufA, gsemA)
            scatter(NB - 1, k, bufA, ssemA)
        for k in range(DEPTH):
            wait_scatter(k, bufB, ssemB)
        for k in range(DEPTH):
            wait_scatter(k, bufA, ssemA)

        plsc.subcore_barrier()
        pltpu.sync_copy(acc.at[pl.ds(base, RSUB)],
                        out_hbm.at[cid, pl.ds(base, RSUB)])

    return agg_kernel(y, src_r, dst_r, zeros_hbm)


_BN = 1024
_GRID = (N_PAD // _BN,)


def _dot(a, b):
    return lax.dot_general(a, b, (((1,), (0,)), ((), ())),
                           precision=lax.Precision.HIGHEST,
                           preferred_element_type=jnp.float32)


def _tc_mm1(x, W1):
    def body(x_ref, w_ref, o_ref):
        o_ref[...] = _dot(x_ref[...], w_ref[...])

    return pl.pallas_call(
        body,
        grid=_GRID,
        in_specs=[pl.BlockSpec((_BN, D), lambda i: (i, 0)),
                  pl.BlockSpec((D, D), lambda i: (0, 0))],
        out_specs=pl.BlockSpec((_BN, D), lambda i: (i, 0)),
        out_shape=jax.ShapeDtypeStruct((N, D), jnp.float32),
    )(x, W1)


def _tc_prep(hists, xw1):
    def body(h_ref, xw_ref, n_ref, y_ref):
        deg = jnp.sum(h_ref[...], axis=(0, 1))
        norm = lax.rsqrt(jnp.maximum(deg, 1.0))
        ones = jnp.ones((1, D), jnp.float32)
        for a in range(8):
            n_ref[pl.ds(a * D, D), :] = lax.dot_general(
                norm[a:a + 1, :], ones, (((0,), (0,)), ((), ())),
                precision=lax.Precision.HIGHEST,
                preferred_element_type=jnp.float32)
        y_ref[...] = xw_ref[...] * n_ref[...]

    return pl.pallas_call(
        body,
        grid=(HR // 8,),
        in_specs=[pl.BlockSpec((NC, NS, 8, D), lambda r: (0, 0, r, 0)),
                  pl.BlockSpec((8 * D, D), lambda r: (r, 0))],
        out_specs=[pl.BlockSpec((8 * D, D), lambda r: (r, 0)),
                   pl.BlockSpec((8 * D, D), lambda r: (r, 0))],
        out_shape=[jax.ShapeDtypeStruct((N_PAD, D), jnp.float32),
                   jax.ShapeDtypeStruct((N, D), jnp.float32)],
    )(hists, xw1)


def _tc_mid(normb, sp, W2):
    def body(n_ref, p_ref, w_ref, o_ref):
        norm = n_ref[...]
        h = jnp.maximum((p_ref[0] + p_ref[1]) * norm, 0.0)
        o_ref[...] = _dot(h * norm, w_ref[...])

    return pl.pallas_call(
        body,
        grid=_GRID,
        in_specs=[pl.BlockSpec((_BN, D), lambda i: (i, 0)),
                  pl.BlockSpec((NC, _BN, D), lambda i: (0, i, 0)),
                  pl.BlockSpec((D, D), lambda i: (0, 0))],
        out_specs=pl.BlockSpec((_BN, D), lambda i: (i, 0)),
        out_shape=jax.ShapeDtypeStruct((N, D), jnp.float32),
    )(normb, sp, W2)


def _tc_final(normb, sp):
    def body(n_ref, p_ref, o_ref):
        o_ref[...] = (p_ref[0] + p_ref[1]) * n_ref[...]

    return pl.pallas_call(
        body,
        grid=_GRID,
        in_specs=[pl.BlockSpec((_BN, D), lambda i: (i, 0)),
                  pl.BlockSpec((NC, _BN, D), lambda i: (0, i, 0))],
        out_specs=pl.BlockSpec((_BN, D), lambda i: (i, 0)),
        out_shape=jax.ShapeDtypeStruct((N, D), jnp.float32),
    )(normb, sp)


def kernel(t, x, edge_index, W1, W2):
    src_r = edge_index[0].reshape(NW, EPW)
    dst_r = edge_index[1].reshape(NW, EPW)
    zerosD = jnp.zeros((RSUB, D), jnp.float32)

    hists = _sc_degree(dst_r, zerosD)
    xw1 = _tc_mm1(x, W1)
    normb, y1 = _tc_prep(hists, xw1)
    s1p = _sc_aggregate(y1, src_r, dst_r, zerosD)
    y2 = _tc_mid(normb, s1p, W2)
    s2p = _sc_aggregate(y2, src_r, dst_r, zerosD)
    return _tc_final(normb, s2p)
